# baseline (speedup 1.0000x reference)
import functools

import jax
import jax.numpy as jnp
from jax import lax
from jax.experimental import pallas as pl
from jax.experimental.pallas import tpu as pltpu
from jax.experimental.pallas import tpu_sc as plsc

NC = 2
NS = 16
NW = NC * NS
LANES = 16

CHUNK = 80
SUB = 40
IB = 25


def _sc_aggregate(src, dst, x, edge_weight, *, n, n_chunks):
    d = edge_weight.shape[1]
    rows_per_tile = -(-n // NS)
    rows_per_tile += (-rows_per_tile) % CHUNK
    n_pad = NS * rows_per_tile
    n_stage = rows_per_tile // CHUNK
    n_iblocks = n_chunks // IB
    pairs = (IB - 1) // 2

    mesh = plsc.VectorSubcoreMesh(
        core_axis_name="c", subcore_axis_name="s", num_cores=NC, num_subcores=NS
    )

    @functools.partial(
        pl.kernel,
        out_type=jax.ShapeDtypeStruct((NC, n_pad, d), jnp.float32),
        mesh=mesh,
        scratch_types=[
            pltpu.VMEM((IB, CHUNK), jnp.int32),
            pltpu.VMEM((IB, CHUNK), jnp.int32),
            pltpu.VMEM((CHUNK, d), jnp.float32),
            pltpu.VMEM((CHUNK, d), jnp.float32),
            pltpu.VMEM((CHUNK, d), jnp.float32),
            pltpu.VMEM_SHARED((n_pad, d), jnp.float32),
            pltpu.SemaphoreType.DMA,
            pltpu.SemaphoreType.DMA,
            pltpu.SemaphoreType.DMA,
            pltpu.SemaphoreType.DMA,
            pltpu.SemaphoreType.DMA,
            pltpu.SemaphoreType.DMA,
        ],
    )
    def agg_kernel(src_hbm, dst_hbm, x_hbm, ew_hbm, out_hbm,
                   src_v, dst_v, xb, eb0, eb1, agg_sh,
                   sx0, sx1, se0, se1, ss0, ss1):
        c = lax.axis_index("c")
        s = lax.axis_index("s")
        wid = s * NC + c
        edge_base = wid * (n_chunks * CHUNK)
        ebufs = [eb0, eb1]
        ses, sss = [se0, se1], [ss0, ss1]
        sx = sx0

        zero = jnp.zeros((LANES,), jnp.float32)

        def zero_row(r, _):
            for cc in range(d // LANES):
                eb0[r, pl.ds(cc * LANES, LANES)] = zero
            return 0

        lax.fori_loop(0, CHUNK, zero_row, 0)
        for k in range(n_stage):
            pltpu.sync_copy(
                eb0, agg_sh.at[pl.ds(s * rows_per_tile + k * CHUNK, CHUNK)]
            )
        plsc.subcore_barrier()

        def ew_desc(ob, j, p):
            off = edge_base + (ob * IB + j) * CHUNK
            return pltpu.make_async_copy(
                ew_hbm.at[pl.ds(off, CHUNK)], ebufs[p], ses[p]
            )

        def gather_desc(j):
            return pltpu.make_async_copy(x_hbm.at[src_v.at[j]], xb, sx)

        def issue_scatter(j, p):
            pltpu.async_copy(ebufs[p], agg_sh.at[dst_v.at[j]], sss[p], add=True)

        def wait_scatter(j, p):
            pltpu.make_async_copy(ebufs[p], agg_sh.at[dst_v.at[j]], sss[p]).wait()

        def compute(p):
            eb = ebufs[p]

            @plsc.parallel_loop(0, CHUNK, unroll=2)
            def _(r):
                for cc in range(d // LANES):
                    sl = pl.ds(cc * LANES, LANES)
                    eb[r, sl] = eb[r, sl] * xb[r, sl]

        def block_body(ob, _):
            pltpu.sync_copy(src_hbm.at[wid, ob], src_v)
            pltpu.sync_copy(dst_hbm.at[wid, ob], dst_v)
            ew_desc(ob, 0, 0).start()
            gather_desc(0).start()

            def compute_steps(ob_, j, p):
                gather_desc(j).wait()
                ew_desc(ob_, j, p).wait()
                compute(p)

                @pl.when(j < IB - 1)
                def _():
                    gather_desc(j + 1).start()
                issue_scatter(j, p)

            def pair_body(pr, _):
                for b in (0, 1):
                    j = pr * 2 + b
                    if b == 0:
                        @pl.when(pr > 0)
                        def _():
                            wait_scatter(j - 1, 1)
                        ew_desc(ob, j + 1, 1).start()
                        compute_steps(ob, j, 0)
                    else:
                        wait_scatter(j - 1, 0)
                        ew_desc(ob, j + 1, 0).start()
                        compute_steps(ob, j, 1)
                return 0

            lax.fori_loop(0, pairs, pair_body, 0)
            j = IB - 1
            wait_scatter(j - 1, 1)
            compute_steps(ob, j, 0)
            wait_scatter(j, 0)
            return 0

        lax.fori_loop(0, n_iblocks, block_body, 0)
        plsc.subcore_barrier()

        for k in range(n_stage):
            rows = pl.ds(s * rows_per_tile + k * CHUNK, CHUNK)
            pltpu.sync_copy(agg_sh.at[rows], eb0)
            pltpu.sync_copy(eb0, out_hbm.at[c].at[rows])

    return agg_kernel(src, dst, x, edge_weight)


def _tc_linear(partials, x, w_l, w_r, b_l, *, block_rows=400):
    n, d = x.shape

    def body(p_ref, x_ref, wl_ref, wr_ref, b_ref, o_ref):
        a = p_ref[0] + p_ref[1]
        o_ref[...] = (
            jnp.dot(a, wl_ref[...], preferred_element_type=jnp.float32)
            + jnp.dot(x_ref[...], wr_ref[...], preferred_element_type=jnp.float32)
            + b_ref[...]
        )

    return pl.pallas_call(
        body,
        grid=(n // block_rows,),
        in_specs=[
            pl.BlockSpec((2, block_rows, d), lambda i: (0, i, 0)),
            pl.BlockSpec((block_rows, d), lambda i: (i, 0)),
            pl.BlockSpec((d, d), lambda i: (0, 0)),
            pl.BlockSpec((d, d), lambda i: (0, 0)),
            pl.BlockSpec((1, d), lambda i: (0, 0)),
        ],
        out_specs=pl.BlockSpec((block_rows, d), lambda i: (i, 0)),
        out_shape=jax.ShapeDtypeStruct((n, d), jnp.float32),
    )(partials, x, w_l, w_r, b_l)


def kernel(x, edge_index, edge_weight, W_l, b_l, W_r):
    n, d = x.shape
    e = edge_weight.shape[0]
    edges_per_tile = e // NW
    n_chunks = edges_per_tile // CHUNK

    src = edge_index[0].astype(jnp.int32).reshape(NW, n_chunks // IB, IB, CHUNK)
    dst = edge_index[1].astype(jnp.int32).reshape(NW, n_chunks // IB, IB, CHUNK)

    partials = _sc_aggregate(src, dst, x, edge_weight, n=n, n_chunks=n_chunks)
    return _tc_linear(partials, x, W_l, W_r, b_l.reshape(1, d))


# speedup vs baseline: 1.2083x; 1.2083x over previous
import functools

import jax
import jax.numpy as jnp
from jax import lax
from jax.experimental import pallas as pl
from jax.experimental.pallas import tpu as pltpu
from jax.experimental.pallas import tpu_sc as plsc

NC = 2
NS = 16
NW = NC * NS
LANES = 16

CHUNK = 80
SUB = 40
IB = 25


def _sc_aggregate(src, dst, x, edge_weight, *, n, n_chunks):
    d = edge_weight.shape[1]
    rows_per_tile = -(-n // NS)
    rows_per_tile += (-rows_per_tile) % CHUNK
    n_pad = NS * rows_per_tile
    n_stage = rows_per_tile // CHUNK
    n_iblocks = n_chunks // IB
    pairs = (IB - 1) // 2

    mesh = plsc.VectorSubcoreMesh(
        core_axis_name="c", subcore_axis_name="s", num_cores=NC, num_subcores=NS
    )

    @functools.partial(
        pl.kernel,
        out_type=jax.ShapeDtypeStruct((NC, n_pad, d), jnp.float32),
        mesh=mesh,
        scratch_types=[
            pltpu.VMEM((IB, 2, SUB), jnp.int32),
            pltpu.VMEM((IB, CHUNK), jnp.int32),
            pltpu.VMEM((SUB, d), jnp.float32),
            pltpu.VMEM((SUB, d), jnp.float32),
            pltpu.VMEM((CHUNK, d), jnp.float32),
            pltpu.VMEM((CHUNK, d), jnp.float32),
            pltpu.VMEM_SHARED((n_pad, d), jnp.float32),
            pltpu.SemaphoreType.DMA,
            pltpu.SemaphoreType.DMA,
            pltpu.SemaphoreType.DMA,
            pltpu.SemaphoreType.DMA,
            pltpu.SemaphoreType.DMA,
            pltpu.SemaphoreType.DMA,
        ],
    )
    def agg_kernel(src_hbm, dst_hbm, x_hbm, ew_hbm, out_hbm,
                   src_v, dst_v, xb0, xb1, eb0, eb1, agg_sh,
                   sx0, sx1, se0, se1, ss0, ss1):
        c = lax.axis_index("c")
        s = lax.axis_index("s")
        wid = s * NC + c
        edge_base = wid * (n_chunks * CHUNK)
        xbufs, ebufs = [xb0, xb1], [eb0, eb1]
        sxs, ses, sss = [sx0, sx1], [se0, se1], [ss0, ss1]

        zero = jnp.zeros((LANES,), jnp.float32)

        def zero_row(r, _):
            for cc in range(d // LANES):
                eb0[r, pl.ds(cc * LANES, LANES)] = zero
            return 0

        lax.fori_loop(0, CHUNK, zero_row, 0)
        for k in range(n_stage):
            pltpu.sync_copy(
                eb0, agg_sh.at[pl.ds(s * rows_per_tile + k * CHUNK, CHUNK)]
            )
        plsc.subcore_barrier()

        def ew_desc(ob, j, p):
            off = edge_base + (ob * IB + j) * CHUNK
            return pltpu.make_async_copy(
                ew_hbm.at[pl.ds(off, CHUNK)], ebufs[p], ses[p]
            )

        def gather_desc(j, h):
            return pltpu.make_async_copy(
                x_hbm.at[src_v.at[j, h]], xbufs[h], sxs[h]
            )

        def issue_scatter(j, p):
            pltpu.async_copy(ebufs[p], agg_sh.at[dst_v.at[j]], sss[p], add=True)

        def wait_scatter(j, p):
            pltpu.make_async_copy(ebufs[p], agg_sh.at[dst_v.at[j]], sss[p]).wait()

        def compute(p, h):
            eb, xb = ebufs[p], xbufs[h]
            base = h * SUB

            @plsc.parallel_loop(0, SUB, unroll=2)
            def _(r):
                for cc in range(d // LANES):
                    sl = pl.ds(cc * LANES, LANES)
                    eb[base + r, sl] = eb[base + r, sl] * xb[r, sl]

        def block_body(ob, _):
            pltpu.sync_copy(src_hbm.at[wid, ob], src_v)
            pltpu.sync_copy(dst_hbm.at[wid, ob], dst_v)
            ew_desc(ob, 0, 0).start()
            gather_desc(0, 0).start()
            gather_desc(0, 1).start()

            def compute_steps(ob_, j, p):
                gather_desc(j, 0).wait()
                ew_desc(ob_, j, p).wait()
                compute(p, 0)
                gather_desc(j + 1, 0).start()
                gather_desc(j, 1).wait()
                compute(p, 1)
                gather_desc(j + 1, 1).start()
                issue_scatter(j, p)

            def pair_body(pr, _):
                for b in (0, 1):
                    j = pr * 2 + b
                    if b == 0:
                        @pl.when(pr > 0)
                        def _():
                            wait_scatter(j - 1, 1)
                        ew_desc(ob, j + 1, 1).start()
                        compute_steps(ob, j, 0)
                    else:
                        wait_scatter(j - 1, 0)
                        ew_desc(ob, j + 1, 0).start()
                        compute_steps(ob, j, 1)
                return 0

            lax.fori_loop(0, pairs, pair_body, 0)
            j = IB - 1
            gather_desc(j, 0).wait()
            ew_desc(ob, j, 0).wait()
            wait_scatter(j - 1, 1)
            compute(0, 0)
            gather_desc(j, 1).wait()
            compute(0, 1)
            issue_scatter(j, 0)
            wait_scatter(j, 0)
            return 0

        lax.fori_loop(0, n_iblocks, block_body, 0)
        plsc.subcore_barrier()

        for k in range(n_stage):
            rows = pl.ds(s * rows_per_tile + k * CHUNK, CHUNK)
            pltpu.sync_copy(agg_sh.at[rows], eb0)
            pltpu.sync_copy(eb0, out_hbm.at[c].at[rows])

    return agg_kernel(src, dst, x, edge_weight)


def _tc_linear(partials, x, w_l, w_r, b_l, *, block_rows=400):
    n, d = x.shape

    def body(p_ref, x_ref, wl_ref, wr_ref, b_ref, o_ref):
        a = p_ref[0] + p_ref[1]
        o_ref[...] = (
            jnp.dot(a, wl_ref[...], preferred_element_type=jnp.float32)
            + jnp.dot(x_ref[...], wr_ref[...], preferred_element_type=jnp.float32)
            + b_ref[...]
        )

    return pl.pallas_call(
        body,
        grid=(n // block_rows,),
        in_specs=[
            pl.BlockSpec((2, block_rows, d), lambda i: (0, i, 0)),
            pl.BlockSpec((block_rows, d), lambda i: (i, 0)),
            pl.BlockSpec((d, d), lambda i: (0, 0)),
            pl.BlockSpec((d, d), lambda i: (0, 0)),
            pl.BlockSpec((1, d), lambda i: (0, 0)),
        ],
        out_specs=pl.BlockSpec((block_rows, d), lambda i: (i, 0)),
        out_shape=jax.ShapeDtypeStruct((n, d), jnp.float32),
    )(partials, x, w_l, w_r, b_l)


def kernel(x, edge_index, edge_weight, W_l, b_l, W_r):
    n, d = x.shape
    e = edge_weight.shape[0]
    edges_per_tile = e // NW
    n_chunks = edges_per_tile // CHUNK

    src = edge_index[0].astype(jnp.int32).reshape(NW, n_chunks // IB, IB, 2, SUB)
    dst = edge_index[1].astype(jnp.int32).reshape(NW, n_chunks // IB, IB, CHUNK)

    partials = _sc_aggregate(src, dst, x, edge_weight, n=n, n_chunks=n_chunks)
    return _tc_linear(partials, x, W_l, W_r, b_l.reshape(1, d))
